# baseline (speedup 1.0000x reference)
import functools

import jax
import jax.numpy as jnp
from jax import lax
from jax.experimental import pallas as pl
from jax.experimental.pallas import tpu as pltpu
from jax.experimental.pallas import tpu_sc as plsc

N_NODES = 50000
N_EDGES = 800000
D = 64
H = 32

NC = 2
NS = 16
CH = 128

N_PAD = 50048
ROWS_PER_TILE = N_PAD // NS
E_PAD = 819200
CHUNKS_PER_TILE = E_PAD // (NS * CH)

N_CHUNKS_EDGE = N_EDGES // CH
N_WORKERS = NC * NS
EDGE_ITERS = (N_CHUNKS_EDGE + N_WORKERS - 1) // N_WORKERS

@functools.lru_cache(maxsize=None)
def _mesh():
    return plsc.VectorSubcoreMesh(
        core_axis_name="c", subcore_axis_name="s",
        num_cores=NC, num_subcores=NS)


NBUF = 6
GLAG = 3
IBLK = 8
N_IBLK = CHUNKS_PER_TILE // IBLK
ZR = 256
ZCOPIES = ROWS_PER_TILE // ZR
ZREM = ROWS_PER_TILE - ZCOPIES * ZR


def _sc_segsum_body(hL, hR, srcp2, dstp2, aggL, aggR, acc, svb, dvb, rows,
                    gsem, ssem, isem):
    c = lax.axis_index("c")
    s = lax.axis_index("s")
    tile_row0 = s * CHUNKS_PER_TILE

    def _fetch_blk(blk):
        b3 = blk % 3
        row = tile_row0 + blk * IBLK
        pltpu.async_copy(srcp2.at[pl.ds(row, IBLK)], svb.at[b3], isem.at[b3])
        pltpu.async_copy(dstp2.at[pl.ds(row, IBLK)], dvb.at[b3], isem.at[b3])

    _fetch_blk(0)
    _fetch_blk(1)

    @pl.loop(0, ZR)
    def _zero(r):
        rows[0, r, 0:16] = jnp.zeros((16,), jnp.float32)
        rows[0, r, 16:32] = jnp.zeros((16,), jnp.float32)

    base_r = s * ROWS_PER_TILE
    for j in range(ZCOPIES):
        pltpu.sync_copy(rows.at[0, pl.ds(0, ZR)],
                        acc.at[pl.ds(base_r + j * ZR, ZR)])
    pltpu.sync_copy(rows.at[0, pl.ds(0, ZREM)],
                    acc.at[pl.ds(base_r + ZCOPIES * ZR, ZREM)])
    plsc.subcore_barrier()

    def _gather(i):
        blk3 = (i // IBLK) % 3
        off = i % IBLK
        j = i % NBUF

        @pl.when(c == 0)
        def _():
            pltpu.async_copy(hL.at[svb.at[blk3, off]], rows.at[j],
                             gsem.at[j])

        @pl.when(c == 1)
        def _():
            pltpu.async_copy(hR.at[svb.at[blk3, off]], rows.at[j],
                             gsem.at[j])

    def _scatter(i):
        blk3 = (i // IBLK) % 3
        off = i % IBLK
        j = i % NBUF
        pltpu.make_async_copy(hL.at[pl.ds(0, CH)], rows.at[j],
                              gsem.at[j]).wait()
        pltpu.async_copy(rows.at[j], acc.at[dvb.at[blk3, off]],
                         ssem.at[j], add=True)

    @pl.loop(0, CHUNKS_PER_TILE)
    def _chunk(i):
        @pl.when(i % IBLK == 0)
        def _():
            blk = i // IBLK
            b3 = blk % 3
            pltpu.make_async_copy(srcp2.at[pl.ds(0, IBLK)], svb.at[b3],
                                  isem.at[b3]).wait()
            pltpu.make_async_copy(srcp2.at[pl.ds(0, IBLK)], dvb.at[b3],
                                  isem.at[b3]).wait()

        @pl.when(i % IBLK == IBLK - 2)
        def _():
            blk = i // IBLK

            @pl.when(blk + 2 < N_IBLK)
            def _():
                _fetch_blk(blk + 2)

        @pl.when(i >= NBUF)
        def _():
            j = i % NBUF
            pltpu.make_async_copy(rows.at[j], acc.at[pl.ds(0, CH)],
                                  ssem.at[j]).wait()

        _gather(i)

        @pl.when(i >= GLAG)
        def _():
            _scatter(i - GLAG)

    for k in range(GLAG, 0, -1):
        _scatter(CHUNKS_PER_TILE - k)
    for j in range(NBUF):
        pltpu.make_async_copy(rows.at[j], acc.at[pl.ds(0, CH)],
                              ssem.at[j]).wait()

    plsc.subcore_barrier()

    @pl.when(c == 0)
    def _():
        pltpu.sync_copy(acc.at[pl.ds(s * ROWS_PER_TILE, ROWS_PER_TILE)],
                        aggL.at[pl.ds(s * ROWS_PER_TILE, ROWS_PER_TILE)])

    @pl.when(c == 1)
    def _():
        pltpu.sync_copy(acc.at[pl.ds(s * ROWS_PER_TILE, ROWS_PER_TILE)],
                        aggR.at[pl.ds(s * ROWS_PER_TILE, ROWS_PER_TILE)])


@functools.lru_cache(maxsize=None)
def _sc_segsum():
    return pl.kernel(
        _sc_segsum_body,
        out_type=(jax.ShapeDtypeStruct((N_PAD, H), jnp.float32),
                  jax.ShapeDtypeStruct((N_PAD, H), jnp.float32)),
        mesh=_mesh(),
        scratch_types=[
            pltpu.VMEM_SHARED((N_PAD, H), jnp.float32),
            pltpu.VMEM((3, IBLK, CH), jnp.int32),
            pltpu.VMEM((3, IBLK, CH), jnp.int32),
            pltpu.VMEM((NBUF, CH, H), jnp.float32),
            pltpu.SemaphoreType.DMA((NBUF,)),
            pltpu.SemaphoreType.DMA((NBUF,)),
            pltpu.SemaphoreType.DMA((3,)),
        ],
        compiler_params=pltpu.CompilerParams(use_tc_tiling_on_sc=False),
    )


EDGE_BASE = N_CHUNKS_EDGE // N_WORKERS
EDGE_XTRA = N_CHUNKS_EDGE - EDGE_BASE * N_WORKERS


def _sc_edge_body(P, Q, srcu2, dstu2, efT, ivs, ivd, bp2, bq2, bpt,
                  gsem, osem):
    c = lax.axis_index("c")
    s = lax.axis_index("s")
    w = s * NC + c
    start = w * EDGE_BASE + jnp.minimum(w, EDGE_XTRA)
    n = EDGE_BASE + jnp.where(w < EDGE_XTRA, 1, 0)
    riota = lax.iota(jnp.int32, 16)

    pltpu.sync_copy(srcu2.at[pl.ds(start, EDGE_BASE + 1)], ivs)
    pltpu.sync_copy(dstu2.at[pl.ds(start, EDGE_BASE + 1)], ivd)

    def _fire(i):
        b = i % 2
        pltpu.async_copy(P.at[ivs.at[i]], bp2.at[b], gsem.at[b])
        pltpu.async_copy(Q.at[ivd.at[i]], bq2.at[b], gsem.at[b])

    def _finish(i):
        b = i % 2
        pltpu.make_async_copy(P.at[pl.ds(0, CH)], bp2.at[b], gsem.at[b]).wait()
        pltpu.make_async_copy(P.at[pl.ds(0, CH)], bq2.at[b], gsem.at[b]).wait()

        @pl.loop(0, D)
        def _col(d):
            cidx = jnp.full((16,), d, jnp.int32)
            for rg in range(CH // 16):
                ridx = riota + (rg * 16)
                v = (plsc.load_gather(bp2.at[b], [ridx, cidx])
                     + plsc.load_gather(bq2.at[b], [ridx, cidx]))
                bpt[b, d, rg * 16:(rg + 1) * 16] = v

        pltpu.async_copy(bpt.at[b], efT.at[:, pl.ds((start + i) * CH, CH)],
                         osem.at[b])

    _fire(0)

    @pl.loop(1, n)
    def _it(i):
        b = i % 2

        @pl.when(i >= 2)
        def _():
            pltpu.make_async_copy(bpt.at[b], efT.at[:, pl.ds(0, CH)],
                                  osem.at[b]).wait()

        _fire(i)
        _finish(i - 1)

    _finish(n - 1)
    for b in range(2):
        pltpu.make_async_copy(bpt.at[b], efT.at[:, pl.ds(0, CH)],
                              osem.at[b]).wait()


@functools.lru_cache(maxsize=None)
def _sc_edge():
    return pl.kernel(
        _sc_edge_body,
        out_type=jax.ShapeDtypeStruct((D, N_EDGES), jnp.float32),
        mesh=_mesh(),
        scratch_types=[
            pltpu.VMEM((EDGE_BASE + 1, CH), jnp.int32),
            pltpu.VMEM((EDGE_BASE + 1, CH), jnp.int32),
            pltpu.VMEM((2, CH, D), jnp.float32),
            pltpu.VMEM((2, CH, D), jnp.float32),
            pltpu.VMEM((2, D, CH), jnp.float32),
            pltpu.SemaphoreType.DMA((2,)),
            pltpu.SemaphoreType.DMA((2,)),
        ],
        compiler_params=pltpu.CompilerParams(use_tc_tiling_on_sc=False,
                                             needs_layout_passes=False),
    )


_BLK = 2000


def _tc_embed_body(c_ref, w_ref, b_ref, hL_ref, hR_ref):
    c = c_ref[...]
    w = w_ref[...]
    h = c[:, 0:1] * w[0:1, :] + c[:, 1:2] * w[1:2, :] + b_ref[...]
    hL_ref[...] = h[:, :H]
    hR_ref[...] = h[:, H:]


def _tc_embed(coords, W_node, b_node):
    g = N_NODES // _BLK
    return pl.pallas_call(
        _tc_embed_body,
        grid=(g,),
        in_specs=[
            pl.BlockSpec((_BLK, 2), lambda i: (i, 0)),
            pl.BlockSpec((2, D), lambda i: (0, 0)),
            pl.BlockSpec((1, D), lambda i: (0, 0)),
        ],
        out_specs=[
            pl.BlockSpec((_BLK, H), lambda i: (i, 0)),
            pl.BlockSpec((_BLK, H), lambda i: (i, 0)),
        ],
        out_shape=[jax.ShapeDtypeStruct((N_NODES, H), jnp.float32),
                   jax.ShapeDtypeStruct((N_NODES, H), jnp.float32)],
    )(coords, W_node, b_node.reshape(1, D))


def _tc_update_body(hL_ref, hR_ref, aL_ref, aR_ref, w_ref, b_ref,
                    oL_ref, oR_ref):
    agg = jnp.concatenate([aL_ref[...], aR_ref[...]], axis=1)
    z = jnp.dot(agg, w_ref[...], preferred_element_type=jnp.float32) + b_ref[...]
    z = jnp.where(z >= 0, z, 0.01 * z)
    oL_ref[...] = hL_ref[...] + z[:, :H]
    oR_ref[...] = hR_ref[...] + z[:, H:]


def _tc_update(hL, hR, aL, aR, W, b):
    g = N_NODES // _BLK
    return pl.pallas_call(
        _tc_update_body,
        grid=(g,),
        in_specs=[
            pl.BlockSpec((_BLK, H), lambda i: (i, 0)),
            pl.BlockSpec((_BLK, H), lambda i: (i, 0)),
            pl.BlockSpec((_BLK, H), lambda i: (i, 0)),
            pl.BlockSpec((_BLK, H), lambda i: (i, 0)),
            pl.BlockSpec((D, D), lambda i: (0, 0)),
            pl.BlockSpec((1, D), lambda i: (0, 0)),
        ],
        out_specs=[
            pl.BlockSpec((_BLK, H), lambda i: (i, 0)),
            pl.BlockSpec((_BLK, H), lambda i: (i, 0)),
        ],
        out_shape=[jax.ShapeDtypeStruct((N_NODES, H), jnp.float32),
                   jax.ShapeDtypeStruct((N_NODES, H), jnp.float32)],
    )(hL, hR, aL, aR, W, b.reshape(1, D))


def _tc_update_proj_body(hL_ref, hR_ref, aL_ref, aR_ref, w_ref, b_ref,
                         we_ref, be_ref, p_ref, q_ref):
    agg = jnp.concatenate([aL_ref[...], aR_ref[...]], axis=1)
    z = jnp.dot(agg, w_ref[...], preferred_element_type=jnp.float32) + b_ref[...]
    z = jnp.where(z >= 0, z, 0.01 * z)
    h = jnp.concatenate([hL_ref[...], hR_ref[...]], axis=1) + z
    we = we_ref[...]
    p_ref[...] = jnp.dot(h, we[:D], preferred_element_type=jnp.float32) + be_ref[...]
    q_ref[...] = jnp.dot(h, we[D:], preferred_element_type=jnp.float32)


def _tc_update_proj(hL, hR, aL, aR, W, b, W_edge, b_edge):
    g = N_NODES // _BLK
    return pl.pallas_call(
        _tc_update_proj_body,
        grid=(g,),
        in_specs=[
            pl.BlockSpec((_BLK, H), lambda i: (i, 0)),
            pl.BlockSpec((_BLK, H), lambda i: (i, 0)),
            pl.BlockSpec((_BLK, H), lambda i: (i, 0)),
            pl.BlockSpec((_BLK, H), lambda i: (i, 0)),
            pl.BlockSpec((D, D), lambda i: (0, 0)),
            pl.BlockSpec((1, D), lambda i: (0, 0)),
            pl.BlockSpec((2 * D, D), lambda i: (0, 0)),
            pl.BlockSpec((1, D), lambda i: (0, 0)),
        ],
        out_specs=[
            pl.BlockSpec((_BLK, D), lambda i: (i, 0)),
            pl.BlockSpec((_BLK, D), lambda i: (i, 0)),
        ],
        out_shape=[jax.ShapeDtypeStruct((N_NODES, D), jnp.float32),
                   jax.ShapeDtypeStruct((N_NODES, D), jnp.float32)],
    )(hL, hR, aL, aR, W, b.reshape(1, D), W_edge, b_edge.reshape(1, D))


def kernel(coords, edge_index, W_node, b_node, W_g0, b_g0, W_g1, b_g1,
           W_g2, b_g2, W_edge, b_edge):
    src = edge_index[0]
    dst = edge_index[1]
    pad = E_PAD - N_EDGES
    srcp2 = jnp.concatenate([src, jnp.zeros((pad,), jnp.int32)]).reshape(-1, CH)
    dstp2 = jnp.concatenate([dst, jnp.full((pad,), N_NODES, jnp.int32)]
                            ).reshape(-1, CH)
    zpad = jnp.zeros((2, CH), jnp.int32)
    srcu2 = jnp.concatenate([src.reshape(-1, CH), zpad])
    dstu2 = jnp.concatenate([dst.reshape(-1, CH), zpad])

    hL, hR = _tc_embed(coords, W_node, b_node)
    for W, b in ((W_g0, b_g0), (W_g1, b_g1)):
        aL, aR = _sc_segsum()(hL, hR, srcp2, dstp2)
        hL, hR = _tc_update(hL, hR, aL, aR, W, b)
    aL, aR = _sc_segsum()(hL, hR, srcp2, dstp2)
    P, Q = _tc_update_proj(hL, hR, aL, aR, W_g2, b_g2, W_edge, b_edge)
    return _sc_edge()(P, Q, srcu2, dstu2).T


# speedup vs baseline: 3.5129x; 3.5129x over previous
import functools

import jax
import jax.numpy as jnp
from jax import lax
from jax.experimental import pallas as pl
from jax.experimental.pallas import tpu as pltpu
from jax.experimental.pallas import tpu_sc as plsc

N_NODES = 50000
N_EDGES = 800000
D = 64
H = 32

NC = 2
NS = 16
CH = 128

N_PAD = 50048
ROWS_PER_TILE = N_PAD // NS
E_PAD = 819200
CHUNKS_PER_TILE = E_PAD // (NS * CH)

N_CHUNKS_EDGE = N_EDGES // CH
N_WORKERS = NC * NS
EDGE_ITERS = (N_CHUNKS_EDGE + N_WORKERS - 1) // N_WORKERS

@functools.lru_cache(maxsize=None)
def _mesh():
    return plsc.VectorSubcoreMesh(
        core_axis_name="c", subcore_axis_name="s",
        num_cores=NC, num_subcores=NS)


NBUF = 6
GLAG = 3
IBLK = 8
N_IBLK = CHUNKS_PER_TILE // IBLK
ZR = 256
ZCOPIES = ROWS_PER_TILE // ZR
ZREM = ROWS_PER_TILE - ZCOPIES * ZR


def _sc_segsum_body(hL, hR, srcp2, dstp2, aggL, aggR, acc, svb, dvb, rows,
                    gsem, ssem, isem):
    c = lax.axis_index("c")
    s = lax.axis_index("s")
    tile_row0 = s * CHUNKS_PER_TILE

    def _fetch_blk(blk):
        b3 = blk % 3
        row = tile_row0 + blk * IBLK
        pltpu.async_copy(srcp2.at[pl.ds(row, IBLK)], svb.at[b3], isem.at[b3])
        pltpu.async_copy(dstp2.at[pl.ds(row, IBLK)], dvb.at[b3], isem.at[b3])

    _fetch_blk(0)
    _fetch_blk(1)

    @pl.loop(0, ZR)
    def _zero(r):
        rows[0, r, 0:16] = jnp.zeros((16,), jnp.float32)
        rows[0, r, 16:32] = jnp.zeros((16,), jnp.float32)

    base_r = s * ROWS_PER_TILE
    for j in range(ZCOPIES):
        pltpu.sync_copy(rows.at[0, pl.ds(0, ZR)],
                        acc.at[pl.ds(base_r + j * ZR, ZR)])
    pltpu.sync_copy(rows.at[0, pl.ds(0, ZREM)],
                    acc.at[pl.ds(base_r + ZCOPIES * ZR, ZREM)])
    plsc.subcore_barrier()

    def _gather(i):
        blk3 = (i // IBLK) % 3
        off = i % IBLK
        j = i % NBUF

        @pl.when(c == 0)
        def _():
            pltpu.async_copy(hL.at[svb.at[blk3, off]], rows.at[j],
                             gsem.at[j])

        @pl.when(c == 1)
        def _():
            pltpu.async_copy(hR.at[svb.at[blk3, off]], rows.at[j],
                             gsem.at[j])

    def _scatter(i):
        blk3 = (i // IBLK) % 3
        off = i % IBLK
        j = i % NBUF
        pltpu.make_async_copy(hL.at[pl.ds(0, CH)], rows.at[j],
                              gsem.at[j]).wait()
        pltpu.async_copy(rows.at[j], acc.at[dvb.at[blk3, off]],
                         ssem.at[j], add=True)

    @pl.loop(0, CHUNKS_PER_TILE)
    def _chunk(i):
        @pl.when(i % IBLK == 0)
        def _():
            blk = i // IBLK
            b3 = blk % 3
            pltpu.make_async_copy(srcp2.at[pl.ds(0, IBLK)], svb.at[b3],
                                  isem.at[b3]).wait()
            pltpu.make_async_copy(srcp2.at[pl.ds(0, IBLK)], dvb.at[b3],
                                  isem.at[b3]).wait()

        @pl.when(i % IBLK == IBLK - 2)
        def _():
            blk = i // IBLK

            @pl.when(blk + 2 < N_IBLK)
            def _():
                _fetch_blk(blk + 2)

        @pl.when(i >= NBUF)
        def _():
            j = i % NBUF
            pltpu.make_async_copy(rows.at[j], acc.at[pl.ds(0, CH)],
                                  ssem.at[j]).wait()

        _gather(i)

        @pl.when(i >= GLAG)
        def _():
            _scatter(i - GLAG)

    for k in range(GLAG, 0, -1):
        _scatter(CHUNKS_PER_TILE - k)
    for j in range(NBUF):
        pltpu.make_async_copy(rows.at[j], acc.at[pl.ds(0, CH)],
                              ssem.at[j]).wait()

    plsc.subcore_barrier()

    @pl.when(c == 0)
    def _():
        pltpu.sync_copy(acc.at[pl.ds(s * ROWS_PER_TILE, ROWS_PER_TILE)],
                        aggL.at[pl.ds(s * ROWS_PER_TILE, ROWS_PER_TILE)])

    @pl.when(c == 1)
    def _():
        pltpu.sync_copy(acc.at[pl.ds(s * ROWS_PER_TILE, ROWS_PER_TILE)],
                        aggR.at[pl.ds(s * ROWS_PER_TILE, ROWS_PER_TILE)])


@functools.lru_cache(maxsize=None)
def _sc_segsum():
    return pl.kernel(
        _sc_segsum_body,
        out_type=(jax.ShapeDtypeStruct((N_PAD, H), jnp.float32),
                  jax.ShapeDtypeStruct((N_PAD, H), jnp.float32)),
        mesh=_mesh(),
        scratch_types=[
            pltpu.VMEM_SHARED((N_PAD, H), jnp.float32),
            pltpu.VMEM((3, IBLK, CH), jnp.int32),
            pltpu.VMEM((3, IBLK, CH), jnp.int32),
            pltpu.VMEM((NBUF, CH, H), jnp.float32),
            pltpu.SemaphoreType.DMA((NBUF,)),
            pltpu.SemaphoreType.DMA((NBUF,)),
            pltpu.SemaphoreType.DMA((3,)),
        ],
        compiler_params=pltpu.CompilerParams(use_tc_tiling_on_sc=False),
    )


EC = 2 * CH
N_STEPS_EDGE = N_EDGES // EC
EDGE_BASE = N_STEPS_EDGE // N_WORKERS
EDGE_XTRA = N_STEPS_EDGE - EDGE_BASE * N_WORKERS


def _sc_edge_body(P, Q, srcu2, dstu2, ef, ivs, ivd, bp2, bq2, gsem, osem):
    c = lax.axis_index("c")
    s = lax.axis_index("s")
    w = s * NC + c
    start = w * EDGE_BASE + jnp.minimum(w, EDGE_XTRA)
    n = EDGE_BASE + jnp.where(w < EDGE_XTRA, 1, 0)

    pltpu.sync_copy(srcu2.at[pl.ds(start * 2, 2 * (EDGE_BASE + 1))], ivs)
    pltpu.sync_copy(dstu2.at[pl.ds(start * 2, 2 * (EDGE_BASE + 1))], ivd)

    def _fire(i):
        b = i % 2
        pltpu.async_copy(P.at[ivs.at[2 * i]],
                         bp2.at[b, pl.ds(0, CH)], gsem.at[b])
        pltpu.async_copy(P.at[ivs.at[2 * i + 1]],
                         bp2.at[b, pl.ds(CH, CH)], gsem.at[b])
        pltpu.async_copy(Q.at[ivd.at[2 * i]],
                         bq2.at[b, pl.ds(0, CH)], gsem.at[b])
        pltpu.async_copy(Q.at[ivd.at[2 * i + 1]],
                         bq2.at[b, pl.ds(CH, CH)], gsem.at[b])

    def _finish(i):
        b = i % 2
        pltpu.make_async_copy(P.at[pl.ds(0, EC)], bp2.at[b], gsem.at[b]).wait()
        pltpu.make_async_copy(P.at[pl.ds(0, EC)], bq2.at[b], gsem.at[b]).wait()

        @pl.loop(0, EC, unroll=8)
        def _row(r):
            for k in range(D // 16):
                bp2[b, r, k * 16:(k + 1) * 16] = (
                    bp2[b, r, k * 16:(k + 1) * 16]
                    + bq2[b, r, k * 16:(k + 1) * 16])

        pltpu.async_copy(bp2.at[b], ef.at[pl.ds((start + i) * EC, EC)],
                         osem.at[b])

    _fire(0)

    @pl.loop(1, n)
    def _it(i):
        b = i % 2

        @pl.when(i >= 2)
        def _():
            pltpu.make_async_copy(bp2.at[b], ef.at[pl.ds(0, EC)],
                                  osem.at[b]).wait()

        _fire(i)
        _finish(i - 1)

    _finish(n - 1)
    for b in range(2):
        pltpu.make_async_copy(bp2.at[b], ef.at[pl.ds(0, EC)], osem.at[b]).wait()


@functools.lru_cache(maxsize=None)
def _sc_edge():
    return pl.kernel(
        _sc_edge_body,
        out_type=jax.ShapeDtypeStruct((N_EDGES, D), jnp.float32),
        mesh=_mesh(),
        scratch_types=[
            pltpu.VMEM((2 * (EDGE_BASE + 1), CH), jnp.int32),
            pltpu.VMEM((2 * (EDGE_BASE + 1), CH), jnp.int32),
            pltpu.VMEM((2, EC, D), jnp.float32),
            pltpu.VMEM((2, EC, D), jnp.float32),
            pltpu.SemaphoreType.DMA((2,)),
            pltpu.SemaphoreType.DMA((2,)),
        ],
        compiler_params=pltpu.CompilerParams(use_tc_tiling_on_sc=False),
    )


_BLK = 2000


def _tc_embed_body(c_ref, w_ref, b_ref, hL_ref, hR_ref):
    c = c_ref[...]
    w = w_ref[...]
    h = c[:, 0:1] * w[0:1, :] + c[:, 1:2] * w[1:2, :] + b_ref[...]
    hL_ref[...] = h[:, :H]
    hR_ref[...] = h[:, H:]


def _tc_embed(coords, W_node, b_node):
    g = N_NODES // _BLK
    return pl.pallas_call(
        _tc_embed_body,
        grid=(g,),
        in_specs=[
            pl.BlockSpec((_BLK, 2), lambda i: (i, 0)),
            pl.BlockSpec((2, D), lambda i: (0, 0)),
            pl.BlockSpec((1, D), lambda i: (0, 0)),
        ],
        out_specs=[
            pl.BlockSpec((_BLK, H), lambda i: (i, 0)),
            pl.BlockSpec((_BLK, H), lambda i: (i, 0)),
        ],
        out_shape=[jax.ShapeDtypeStruct((N_NODES, H), jnp.float32),
                   jax.ShapeDtypeStruct((N_NODES, H), jnp.float32)],
    )(coords, W_node, b_node.reshape(1, D))


def _tc_update_body(hL_ref, hR_ref, aL_ref, aR_ref, w_ref, b_ref,
                    oL_ref, oR_ref):
    agg = jnp.concatenate([aL_ref[...], aR_ref[...]], axis=1)
    z = jnp.dot(agg, w_ref[...], preferred_element_type=jnp.float32) + b_ref[...]
    z = jnp.where(z >= 0, z, 0.01 * z)
    oL_ref[...] = hL_ref[...] + z[:, :H]
    oR_ref[...] = hR_ref[...] + z[:, H:]


def _tc_update(hL, hR, aL, aR, W, b):
    g = N_NODES // _BLK
    return pl.pallas_call(
        _tc_update_body,
        grid=(g,),
        in_specs=[
            pl.BlockSpec((_BLK, H), lambda i: (i, 0)),
            pl.BlockSpec((_BLK, H), lambda i: (i, 0)),
            pl.BlockSpec((_BLK, H), lambda i: (i, 0)),
            pl.BlockSpec((_BLK, H), lambda i: (i, 0)),
            pl.BlockSpec((D, D), lambda i: (0, 0)),
            pl.BlockSpec((1, D), lambda i: (0, 0)),
        ],
        out_specs=[
            pl.BlockSpec((_BLK, H), lambda i: (i, 0)),
            pl.BlockSpec((_BLK, H), lambda i: (i, 0)),
        ],
        out_shape=[jax.ShapeDtypeStruct((N_NODES, H), jnp.float32),
                   jax.ShapeDtypeStruct((N_NODES, H), jnp.float32)],
    )(hL, hR, aL, aR, W, b.reshape(1, D))


def _tc_update_proj_body(hL_ref, hR_ref, aL_ref, aR_ref, w_ref, b_ref,
                         we_ref, be_ref, p_ref, q_ref):
    agg = jnp.concatenate([aL_ref[...], aR_ref[...]], axis=1)
    z = jnp.dot(agg, w_ref[...], preferred_element_type=jnp.float32) + b_ref[...]
    z = jnp.where(z >= 0, z, 0.01 * z)
    h = jnp.concatenate([hL_ref[...], hR_ref[...]], axis=1) + z
    we = we_ref[...]
    p_ref[...] = jnp.dot(h, we[:D], preferred_element_type=jnp.float32) + be_ref[...]
    q_ref[...] = jnp.dot(h, we[D:], preferred_element_type=jnp.float32)


def _tc_update_proj(hL, hR, aL, aR, W, b, W_edge, b_edge):
    g = N_NODES // _BLK
    return pl.pallas_call(
        _tc_update_proj_body,
        grid=(g,),
        in_specs=[
            pl.BlockSpec((_BLK, H), lambda i: (i, 0)),
            pl.BlockSpec((_BLK, H), lambda i: (i, 0)),
            pl.BlockSpec((_BLK, H), lambda i: (i, 0)),
            pl.BlockSpec((_BLK, H), lambda i: (i, 0)),
            pl.BlockSpec((D, D), lambda i: (0, 0)),
            pl.BlockSpec((1, D), lambda i: (0, 0)),
            pl.BlockSpec((2 * D, D), lambda i: (0, 0)),
            pl.BlockSpec((1, D), lambda i: (0, 0)),
        ],
        out_specs=[
            pl.BlockSpec((_BLK, D), lambda i: (i, 0)),
            pl.BlockSpec((_BLK, D), lambda i: (i, 0)),
        ],
        out_shape=[jax.ShapeDtypeStruct((N_NODES, D), jnp.float32),
                   jax.ShapeDtypeStruct((N_NODES, D), jnp.float32)],
    )(hL, hR, aL, aR, W, b.reshape(1, D), W_edge, b_edge.reshape(1, D))


def kernel(coords, edge_index, W_node, b_node, W_g0, b_g0, W_g1, b_g1,
           W_g2, b_g2, W_edge, b_edge):
    src = edge_index[0]
    dst = edge_index[1]
    pad = E_PAD - N_EDGES
    srcp2 = jnp.concatenate([src, jnp.zeros((pad,), jnp.int32)]).reshape(-1, CH)
    dstp2 = jnp.concatenate([dst, jnp.full((pad,), N_NODES, jnp.int32)]
                            ).reshape(-1, CH)
    zpad = jnp.zeros((2, CH), jnp.int32)
    srcu2 = jnp.concatenate([src.reshape(-1, CH), zpad])
    dstu2 = jnp.concatenate([dst.reshape(-1, CH), zpad])

    hL, hR = _tc_embed(coords, W_node, b_node)
    for W, b in ((W_g0, b_g0), (W_g1, b_g1)):
        aL, aR = _sc_segsum()(hL, hR, srcp2, dstp2)
        hL, hR = _tc_update(hL, hR, aL, aR, W, b)
    aL, aR = _sc_segsum()(hL, hR, srcp2, dstp2)
    P, Q = _tc_update_proj(hL, hR, aL, aR, W_g2, b_g2, W_edge, b_edge)
    return _sc_edge()(P, Q, srcu2, dstu2)
